# baseline (speedup 1.0000x reference)
import functools

import jax
import jax.numpy as jnp
from jax import lax
from jax.experimental import pallas as pl
from jax.experimental.pallas import tpu as pltpu
from jax.experimental.pallas import tpu_sc as plsc

_R = 12
_P = 2 * _R + 1
_NSH = _P * _P
_NSEG = 40
_K = 20
_NCLS = 32
_HW = 64
_CH = 256
_PW = _HW + 2 * _R
_QH = 2304
_FRROWS = _PW * _PW + 64
_NEG = -1e30


def _corr_body(ftT_ref, frpT_ref, out_ref):
    p = pl.program_id(0)
    a2 = ftT_ref[0]
    h2 = frpT_ref[pl.ds(p * 2 * _PW, _QH), :]
    mt = lax.dot_general(a2, h2, (((1,), (1,)), ((), ())),
                         preferred_element_type=jnp.float32,
                         precision=lax.Precision.HIGHEST)
    li = lax.broadcasted_iota(jnp.int32, (128, 128), 0)
    ri = lax.broadcasted_iota(jnp.int32, (128, 128), 1)
    hi = ri >= li
    rolls = [pltpu.roll(mt[:, a * 128:(a + 1) * 128], 0, 1,
                        stride=1, stride_axis=0) for a in range(18)]
    skewed = jnp.concatenate(
        [rolls[0]] + [jnp.where(hi, rolls[a], rolls[a - 1])
                      for a in range(1, 18)], axis=1)

    blks = []
    for i in range(_P):
        a = skewed[0:64, i * _PW + 151: i * _PW + 176]
        b = skewed[64:128, i * _PW + 127: i * _PW + 152]
        blks.append(jnp.concatenate([a, b], axis=0))
    out_ref[0] = jnp.concatenate(
        blks + [jnp.full((128, 15), _NEG, jnp.float32)], axis=1)


def _sc_body(corr_hbm, lab_hbm, out_hbm, corr_v, seg_v, sup_v, lab_v, out_v):
    wid = lax.axis_index("s") * 2 + lax.axis_index("c")
    pltpu.sync_copy(corr_hbm.at[wid], corr_v)
    pltpu.sync_copy(lab_hbm, lab_v)

    liota = lax.iota(jnp.int32, 16)
    zeros16 = jnp.zeros(16, jnp.float32)
    for ch in range(_NCLS):
        for b in range(8):
            out_v[pl.ds(ch * 128 + b * 16, 16)] = zeros16

    for g in range(8):
        px = g * 16 + liota
        h = jnp.where(px < 64, 1, 0)
        y = 2 * wid + h
        x = jnp.where(px < 64, 63 - px, 127 - px)

        px640 = px * 640

        def seg_build(s, _):
            m = plsc.load_gather(corr_v, [px640 + s * 16])
            for t in range(1, 16):
                m = jnp.maximum(
                    m, plsc.load_gather(corr_v, [px640 + s * 16 + t]))
            seg_v[pl.ds(s * 128 + g * 16, 16)] = m
            return 0
        lax.fori_loop(0, _NSEG, seg_build, 0)

        def sup_build(u, _):
            m = seg_v[pl.ds(u * 640 + g * 16, 16)]
            for d in range(1, 5):
                m = jnp.maximum(m, seg_v[pl.ds(u * 640 + d * 128 + g * 16, 16)])
            sup_v[pl.ds(u * 128 + g * 16, 16)] = m
            return 0
        lax.fori_loop(0, 8, sup_build, 0)

        def round_body(k, carry):
            v0, z = carry
            best = sup_v[pl.ds(g * 16, 16)]
            ubest = jnp.zeros(16, jnp.int32)
            for u in range(1, 8):
                v = sup_v[pl.ds(u * 128 + g * 16, 16)]
                gt = v > best
                best = jnp.where(gt, v, best)
                ubest = jnp.where(gt, u, ubest)

            sbase = ubest * 5
            sbest = sbase
            best = plsc.load_gather(seg_v, [sbase * 128 + px])
            for d in range(1, 5):
                v = plsc.load_gather(seg_v, [(sbase + d) * 128 + px])
                gt = v > best
                best = jnp.where(gt, v, best)
                sbest = jnp.where(gt, sbase + d, sbest)

            base = sbest * 16
            bv = plsc.load_gather(corr_v, [px640 + base])
            nbest = base
            m2 = jnp.full(16, _NEG, jnp.float32)
            for t in range(1, 16):
                idx = base + t
                v = plsc.load_gather(corr_v, [px640 + idx])
                gt = v > bv
                m2 = jnp.where(gt, bv, jnp.maximum(m2, v))
                bv = jnp.where(gt, v, bv)
                nbest = jnp.where(gt, idx, nbest)

            plsc.store_scatter(corr_v, [px640 + nbest],
                               jnp.full(16, _NEG, jnp.float32))
            plsc.store_scatter(seg_v, [sbest * 128 + px], m2)
            sm = plsc.load_gather(seg_v, [sbase * 128 + px])
            for d in range(1, 5):
                sm = jnp.maximum(
                    sm, plsc.load_gather(seg_v, [(sbase + d) * 128 + px]))
            plsc.store_scatter(sup_v, [ubest * 128 + px], sm)

            i = nbest // _P
            j = nbest - i * _P
            r = jnp.clip(y + i - _R, 0, _HW - 1)
            c = jnp.clip(x + j - _R, 0, _HW - 1)
            lbl = plsc.load_gather(lab_v, [r * 64 + c])

            v0 = jnp.where(k == 0, bv, v0)
            e = jnp.exp(bv - v0)
            plsc.addupdate_scatter(out_v, [lbl * 128 + px], e)
            return v0, z + e

        _, z = lax.fori_loop(0, _K, round_body, (zeros16, zeros16))

        def norm(ch, _):
            vv = out_v[pl.ds(ch * 128 + g * 16, 16)]
            out_v[pl.ds(ch * 128 + g * 16, 16)] = vv / z
            return 0
        lax.fori_loop(0, _NCLS, norm, 0)

    pltpu.sync_copy(out_v, out_hbm.at[wid])


_sc_retrieve = functools.partial(
    pl.kernel,
    out_type=jax.ShapeDtypeStruct((32, _NCLS * 128), jnp.float32),
    mesh=plsc.VectorSubcoreMesh(core_axis_name="c", subcore_axis_name="s"),
    scratch_types=[
        pltpu.VMEM((640 * 128,), jnp.float32),
        pltpu.VMEM((_NSEG * 128,), jnp.float32),
        pltpu.VMEM((8 * 128,), jnp.float32),
        pltpu.VMEM((_HW * _HW,), jnp.int32),
        pltpu.VMEM((_NCLS * 128,), jnp.float32),
    ],
    compiler_params=pltpu.CompilerParams(needs_layout_passes=False),
)(_sc_body)


def kernel(feats_r, feats_t, quantized_r, ref_index, current_ind, dil_int):
    ft = feats_t[0]
    fr = feats_r[0]
    ftT = jnp.transpose(ft, (1, 2, 0)).reshape(32, 128, _CH)[:, ::-1, :]
    frp = jnp.pad(fr, ((0, 0), (_R, _R), (_R, _R)))
    frpT = jnp.transpose(frp, (1, 2, 0)).reshape(_PW * _PW, _CH)
    frpT = jnp.pad(frpT, ((0, 64), (0, 0)))
    labels = quantized_r[0, 0, ::4, ::4].astype(jnp.int32)

    corr = pl.pallas_call(
        _corr_body,
        grid=(32,),
        in_specs=[
            pl.BlockSpec((1, 128, _CH), lambda p: (p, 0, 0)),
            pl.BlockSpec((_FRROWS, _CH), lambda p: (0, 0)),
        ],
        out_specs=pl.BlockSpec((1, 128, 640), lambda p: (p, 0, 0)),
        out_shape=jax.ShapeDtypeStruct((32, 128, 640), jnp.float32),
    )(ftT, frpT)

    out = _sc_retrieve(corr.reshape(32, 640 * 128), labels.reshape(-1))

    out = out.reshape(32, _NCLS, 128)
    out = out[:, :, ::-1].reshape(32, _NCLS, 2, 64).transpose(1, 0, 2, 3).reshape(
        1, _NCLS, _HW, _HW)
    return out


# speedup vs baseline: 1.3134x; 1.3134x over previous
import functools

import jax
import jax.numpy as jnp
from jax import lax
from jax.experimental import pallas as pl
from jax.experimental.pallas import tpu as pltpu
from jax.experimental.pallas import tpu_sc as plsc

_R = 12
_P = 2 * _R + 1
_NSH = _P * _P
_NSEG = 40
_K = 20
_NCLS = 32
_HW = 64
_CH = 256
_PW = _HW + 2 * _R
_QH = 2304
_FRROWS = _PW * _PW + 64
_NEG = -1e30


def _corr_body(ftT_ref, frpT_ref, out_ref):
    p = pl.program_id(0)
    a2 = ftT_ref[0]
    h2 = frpT_ref[pl.ds(p * 2 * _PW, _QH), :]
    dims = (((1,), (1,)), ((), ()))
    a_hi = a2.astype(jnp.bfloat16)
    a_lo = (a2 - a_hi.astype(jnp.float32)).astype(jnp.bfloat16)
    h_hi = h2.astype(jnp.bfloat16)
    h_lo = (h2 - h_hi.astype(jnp.float32)).astype(jnp.bfloat16)
    mt = (lax.dot_general(a_hi, h_hi, dims, preferred_element_type=jnp.float32)
          + lax.dot_general(a_lo, h_hi, dims, preferred_element_type=jnp.float32)
          + lax.dot_general(a_hi, h_lo, dims, preferred_element_type=jnp.float32)
          )
    li = lax.broadcasted_iota(jnp.int32, (128, 128), 0)
    ri = lax.broadcasted_iota(jnp.int32, (128, 128), 1)
    hi = ri >= li
    rolls = [pltpu.roll(mt[:, a * 128:(a + 1) * 128], 0, 1,
                        stride=1, stride_axis=0) for a in range(18)]
    skewed = jnp.concatenate(
        [rolls[0]] + [jnp.where(hi, rolls[a], rolls[a - 1])
                      for a in range(1, 18)], axis=1)

    blks = []
    for i in range(_P):
        a = skewed[0:64, i * _PW + 151: i * _PW + 176]
        b = skewed[64:128, i * _PW + 127: i * _PW + 152]
        blks.append(jnp.concatenate([a, b], axis=0))
    corr_t = jnp.concatenate(
        blks + [jnp.full((128, 15), _NEG, jnp.float32)], axis=1)
    out_ref[0] = jnp.transpose(corr_t, (1, 0))


def _sc_body(corr_hbm, lab_hbm, out_hbm, corr_v, seg_v, sup_v, lab_v, out_v):
    wid = lax.axis_index("s") * 2 + lax.axis_index("c")
    pltpu.sync_copy(corr_hbm.at[wid], corr_v)
    pltpu.sync_copy(lab_hbm, lab_v)

    liota = lax.iota(jnp.int32, 16)
    zeros16 = jnp.zeros(16, jnp.float32)
    for ch in range(_NCLS):
        for b in range(8):
            out_v[pl.ds(ch * 128 + b * 16, 16)] = zeros16

    for g in range(8):
        px = g * 16 + liota
        h = jnp.where(px < 64, 1, 0)
        y = 2 * wid + h
        x = jnp.where(px < 64, 63 - px, 127 - px)

        def seg_build(s, _):
            m = corr_v[pl.ds(s * 2048 + g * 16, 16)]
            for t in range(1, 16):
                m = jnp.maximum(m, corr_v[pl.ds(s * 2048 + t * 128 + g * 16, 16)])
            seg_v[pl.ds(s * 128 + g * 16, 16)] = m
            return 0
        lax.fori_loop(0, _NSEG, seg_build, 0)

        def sup_build(u, _):
            m = seg_v[pl.ds(u * 640 + g * 16, 16)]
            for d in range(1, 5):
                m = jnp.maximum(m, seg_v[pl.ds(u * 640 + d * 128 + g * 16, 16)])
            sup_v[pl.ds(u * 128 + g * 16, 16)] = m
            return 0
        lax.fori_loop(0, 8, sup_build, 0)

        def round_body(k, carry):
            v0, z = carry
            best = sup_v[pl.ds(g * 16, 16)]
            ubest = jnp.zeros(16, jnp.int32)
            for u in range(1, 8):
                v = sup_v[pl.ds(u * 128 + g * 16, 16)]
                gt = v > best
                best = jnp.where(gt, v, best)
                ubest = jnp.where(gt, u, ubest)

            sbase = ubest * 5
            sbest = sbase
            best = plsc.load_gather(seg_v, [sbase * 128 + px])
            for d in range(1, 5):
                v = plsc.load_gather(seg_v, [(sbase + d) * 128 + px])
                gt = v > best
                best = jnp.where(gt, v, best)
                sbest = jnp.where(gt, sbase + d, sbest)

            base = sbest * 16
            bv = plsc.load_gather(corr_v, [base * 128 + px])
            nbest = base
            m2 = jnp.full(16, _NEG, jnp.float32)
            for t in range(1, 16):
                idx = base + t
                v = plsc.load_gather(corr_v, [idx * 128 + px])
                gt = v > bv
                m2 = jnp.where(gt, bv, jnp.maximum(m2, v))
                bv = jnp.where(gt, v, bv)
                nbest = jnp.where(gt, idx, nbest)

            plsc.store_scatter(corr_v, [nbest * 128 + px],
                               jnp.full(16, _NEG, jnp.float32))
            plsc.store_scatter(seg_v, [sbest * 128 + px], m2)
            sm = plsc.load_gather(seg_v, [sbase * 128 + px])
            for d in range(1, 5):
                sm = jnp.maximum(
                    sm, plsc.load_gather(seg_v, [(sbase + d) * 128 + px]))
            plsc.store_scatter(sup_v, [ubest * 128 + px], sm)

            i = nbest // _P
            j = nbest - i * _P
            r = jnp.clip(y + i - _R, 0, _HW - 1)
            c = jnp.clip(x + j - _R, 0, _HW - 1)
            lbl = plsc.load_gather(lab_v, [r * 64 + c])

            v0 = jnp.where(k == 0, bv, v0)
            e = jnp.exp(bv - v0)
            plsc.addupdate_scatter(out_v, [lbl * 128 + px], e)
            return v0, z + e

        _, z = lax.fori_loop(0, _K, round_body, (zeros16, zeros16))

        def norm(ch, _):
            vv = out_v[pl.ds(ch * 128 + g * 16, 16)]
            out_v[pl.ds(ch * 128 + g * 16, 16)] = vv / z
            return 0
        lax.fori_loop(0, _NCLS, norm, 0)

    pltpu.sync_copy(out_v, out_hbm.at[wid])


_sc_retrieve = functools.partial(
    pl.kernel,
    out_type=jax.ShapeDtypeStruct((32, _NCLS * 128), jnp.float32),
    mesh=plsc.VectorSubcoreMesh(core_axis_name="c", subcore_axis_name="s"),
    scratch_types=[
        pltpu.VMEM((640 * 128,), jnp.float32),
        pltpu.VMEM((_NSEG * 128,), jnp.float32),
        pltpu.VMEM((8 * 128,), jnp.float32),
        pltpu.VMEM((_HW * _HW,), jnp.int32),
        pltpu.VMEM((_NCLS * 128,), jnp.float32),
    ],
    compiler_params=pltpu.CompilerParams(needs_layout_passes=False),
)(_sc_body)


def kernel(feats_r, feats_t, quantized_r, ref_index, current_ind, dil_int):
    ft = feats_t[0]
    fr = feats_r[0]
    ftT = jnp.transpose(ft, (1, 2, 0)).reshape(32, 128, _CH)[:, ::-1, :]
    frp = jnp.pad(fr, ((0, 0), (_R, _R), (_R, _R)))
    frpT = jnp.transpose(frp, (1, 2, 0)).reshape(_PW * _PW, _CH)
    frpT = jnp.pad(frpT, ((0, 64), (0, 0)))
    labels = quantized_r[0, 0, ::4, ::4].astype(jnp.int32)

    corr = pl.pallas_call(
        _corr_body,
        grid=(32,),
        in_specs=[
            pl.BlockSpec((1, 128, _CH), lambda p: (p, 0, 0)),
            pl.BlockSpec((_FRROWS, _CH), lambda p: (0, 0)),
        ],
        out_specs=pl.BlockSpec((1, 640, 128), lambda p: (p, 0, 0)),
        out_shape=jax.ShapeDtypeStruct((32, 640, 128), jnp.float32),
    )(ftT, frpT)

    out = _sc_retrieve(corr.reshape(32, 640 * 128), labels.reshape(-1))

    out = out.reshape(32, _NCLS, 128)
    out = out[:, :, ::-1].reshape(32, _NCLS, 2, 64).transpose(1, 0, 2, 3).reshape(
        1, _NCLS, _HW, _HW)
    return out
